# baseline (speedup 1.0000x reference)
import functools

import jax
import jax.numpy as jnp
from jax import lax
from jax.experimental import pallas as pl
from jax.experimental.pallas import tpu as pltpu
from jax.experimental.pallas import tpu_sc as plsc

N = 10000
NPAD = 10240
E = 320000
CH = 3200
NCHUNK = E // CH
FPT = 4
NPASS = 2

_f32 = jnp.float32
_i32 = jnp.int32


def _deg_body(dst_hbm, w_hbm, degp_hbm, ldeg, dstb, wbuf):
    cid = lax.axis_index("c")
    sid = lax.axis_index("s")
    wid = sid * 2 + cid

    def zero_body(i, _):
        ldeg[pl.ds(i * 16, 16)] = jnp.zeros((16,), _f32)
        return 0

    lax.fori_loop(0, NPAD // 16, zero_body, 0)

    per_tile = E // 32
    base = wid * per_tile
    pltpu.sync_copy(dst_hbm.at[pl.ds(base, per_tile)], dstb)
    pltpu.sync_copy(w_hbm.at[pl.ds(base, per_tile)], wbuf)

    def acc_body(s, _):
        dv = dstb[pl.ds(s * 16, 16)]
        wv = wbuf[pl.ds(s * 16, 16)]
        plsc.addupdate_scatter(ldeg, [dv], wv)
        return 0

    lax.fori_loop(0, per_tile // 16, acc_body, 0)
    pltpu.sync_copy(ldeg, degp_hbm.at[wid])


def _norm_body(src_hbm, dst_hbm, w_hbm, dis_hbm, nw_hbm, disb, srcb, dstb, wbuf):
    cid = lax.axis_index("c")
    sid = lax.axis_index("s")
    wid = sid * 2 + cid
    per_tile = E // 32
    base = wid * per_tile

    pltpu.sync_copy(dis_hbm, disb)
    pltpu.sync_copy(src_hbm.at[pl.ds(base, per_tile)], srcb)
    pltpu.sync_copy(dst_hbm.at[pl.ds(base, per_tile)], dstb)
    pltpu.sync_copy(w_hbm.at[pl.ds(base, per_tile)], wbuf)

    def body(s, _):
        sv = srcb[pl.ds(s * 16, 16)]
        dv = dstb[pl.ds(s * 16, 16)]
        wv = wbuf[pl.ds(s * 16, 16)]
        ns = plsc.load_gather(disb, [sv])
        nd = plsc.load_gather(disb, [dv])
        wbuf[pl.ds(s * 16, 16)] = ns * wv * nd
        return 0

    lax.fori_loop(0, per_tile // 16, body, 0)
    pltpu.sync_copy(wbuf, nw_hbm.at[pl.ds(base, per_tile)])


def _prop_body(z_hbm, src_hbm, dst_hbm, w_hbm, dis_hbm, bias_hbm, out_hbm,
               disb, biasb, zb0, zb1, zb2, zb3, ac0, ac1, ac2, ac3,
               srcb0, srcb1, dstb0, dstb1, wb0, wb1, sem0, sem1, semz):
    cid = lax.axis_index("c")
    sid = lax.axis_index("s")
    wid = sid * 2 + cid

    pltpu.sync_copy(dis_hbm, disb)
    pltpu.sync_copy(bias_hbm, biasb)

    zbs = (zb0, zb1, zb2, zb3)
    acs = (ac0, ac1, ac2, ac3)

    srcbs = (srcb0, srcb1)
    dstbs = (dstb0, dstb1)
    wbs = (wb0, wb1)

    def issue(c, slot):
        sem = sem0 if slot == 0 else sem1
        pltpu.async_copy(src_hbm.at[pl.ds(c * CH, CH)], srcbs[slot], sem)
        pltpu.async_copy(dst_hbm.at[pl.ds(c * CH, CH)], dstbs[slot], sem)
        pltpu.async_copy(w_hbm.at[pl.ds(c * CH, CH)], wbs[slot], sem)

    def drain(c, slot):
        sem = sem0 if slot == 0 else sem1
        pltpu.make_async_copy(src_hbm.at[pl.ds(c * CH, CH)], srcbs[slot], sem).wait()
        pltpu.make_async_copy(dst_hbm.at[pl.ds(c * CH, CH)], dstbs[slot], sem).wait()
        pltpu.make_async_copy(w_hbm.at[pl.ds(c * CH, CH)], wbs[slot], sem).wait()

    for p in range(NPASS):
        fb = p * 128 + wid * FPT

        for r in range(FPT):
            pltpu.async_copy(z_hbm.at[fb + r], zbs[r], semz)
        for r in range(FPT):
            pltpu.make_async_copy(z_hbm.at[fb + r], zbs[r], semz).wait()

        bvs = [plsc.load_gather(biasb, [jnp.full((16,), fb + r, _i32)])
               for r in range(FPT)]

        @plsc.parallel_loop(0, N // 16, unroll=4)
        def _init(i):
            dv = disb[pl.ds(i * 16, 16)]
            sn = dv * dv
            for r in range(FPT):
                zv = zbs[r][pl.ds(i * 16, 16)]
                acs[r][pl.ds(i * 16, 16)] = zv * sn + bvs[r]

        issue(0, 0)

        def chunk_pair(g, _):
            for b2 in (0, 1):
                c = g * 2 + b2
                drain(c, b2)

                @pl.when(c + 1 < NCHUNK)
                def _():
                    issue(c + 1, 1 - b2)

                @plsc.parallel_loop(0, CH // 16, unroll=8)
                def _sub(s):
                    sv = srcbs[b2][pl.ds(s * 16, 16)]
                    dv = dstbs[b2][pl.ds(s * 16, 16)]
                    nv = wbs[b2][pl.ds(s * 16, 16)]
                    for r in range(FPT):
                        g_ = plsc.load_gather(zbs[r], [sv])
                        plsc.addupdate_scatter(acs[r], [dv], g_ * nv)
            return 0

        lax.fori_loop(0, NCHUNK // 2, chunk_pair, 0)

        for r in range(FPT):
            pltpu.async_copy(acs[r], out_hbm.at[fb + r], semz)
        for r in range(FPT):
            pltpu.make_async_copy(acs[r], out_hbm.at[fb + r], semz).wait()


_sc_mesh = plsc.VectorSubcoreMesh(core_axis_name="c", subcore_axis_name="s")
_sc_params = pltpu.CompilerParams(needs_layout_passes=False)

_deg = functools.partial(
    pl.kernel,
    mesh=_sc_mesh,
    compiler_params=_sc_params,
    out_type=jax.ShapeDtypeStruct((32, NPAD), _f32),
    scratch_types=[
        pltpu.VMEM((NPAD,), _f32),
        pltpu.VMEM((E // 32,), _i32),
        pltpu.VMEM((E // 32,), _f32),
    ],
)(_deg_body)

_norm = functools.partial(
    pl.kernel,
    mesh=_sc_mesh,
    compiler_params=_sc_params,
    out_type=jax.ShapeDtypeStruct((E,), _f32),
    scratch_types=[
        pltpu.VMEM((NPAD,), _f32),
        pltpu.VMEM((E // 32,), _i32),
        pltpu.VMEM((E // 32,), _i32),
        pltpu.VMEM((E // 32,), _f32),
    ],
)(_norm_body)

_prop = functools.partial(
    pl.kernel,
    mesh=_sc_mesh,
    compiler_params=_sc_params,
    out_type=jax.ShapeDtypeStruct((256, N), _f32),
    scratch_types=[
        pltpu.VMEM((NPAD,), _f32),
        pltpu.VMEM((256,), _f32),
        pltpu.VMEM((N,), _f32), pltpu.VMEM((N,), _f32),
        pltpu.VMEM((N,), _f32), pltpu.VMEM((N,), _f32),
        pltpu.VMEM((N,), _f32), pltpu.VMEM((N,), _f32),
        pltpu.VMEM((N,), _f32), pltpu.VMEM((N,), _f32),
        pltpu.VMEM((CH,), _i32), pltpu.VMEM((CH,), _i32),
        pltpu.VMEM((CH,), _i32), pltpu.VMEM((CH,), _i32),
        pltpu.VMEM((CH,), _f32), pltpu.VMEM((CH,), _f32),
        pltpu.SemaphoreType.DMA,
        pltpu.SemaphoreType.DMA,
        pltpu.SemaphoreType.DMA,
    ],
)(_prop_body)


def _dis_body(degp_ref, dis_ref):
    dis_ref[...] = lax.rsqrt(jnp.sum(degp_ref[...], axis=0) + 1.0)


def _dis(degp):
    return pl.pallas_call(
        _dis_body,
        out_shape=jax.ShapeDtypeStruct((NPAD,), _f32),
    )(degp)


def _mm1_body(w_ref, x_ref, o_ref):
    o_ref[...] = lax.dot_general(
        w_ref[...], x_ref[...], (((0,), (1,)), ((), ())),
        preferred_element_type=_f32)


def _mm2_body(w_ref, h_ref, o_ref):
    h = jnp.maximum(h_ref[...], 0.0)
    o_ref[...] = lax.dot_general(
        w_ref[...], h, (((0,), (0,)), ((), ())),
        preferred_element_type=_f32)


def _mm1(W1, x):
    return pl.pallas_call(
        _mm1_body,
        grid=(2,),
        in_specs=[
            pl.BlockSpec((128, 128), lambda i: (0, i)),
            pl.BlockSpec((N, 128), lambda i: (0, 0)),
        ],
        out_specs=pl.BlockSpec((128, N), lambda i: (i, 0)),
        out_shape=jax.ShapeDtypeStruct((256, N), _f32),
    )(W1, x)


def _mm2(Wc, h1t):
    return pl.pallas_call(
        _mm2_body,
        grid=(2,),
        in_specs=[
            pl.BlockSpec((256, 128), lambda i: (0, i)),
            pl.BlockSpec((256, N), lambda i: (0, 0)),
        ],
        out_specs=pl.BlockSpec((128, N), lambda i: (i, 0)),
        out_shape=jax.ShapeDtypeStruct((256, N), _f32),
    )(Wc, h1t)


def kernel(x, edge_index, edge_attr, W1, b1, Wmu, bmu, Wlv, blv):
    src = edge_index[0]
    dst = edge_index[1]
    w = edge_attr

    degp = _deg(dst, w)
    dis = _dis(degp)
    nw = _norm(src, dst, w, dis)
    z1t = _mm1(W1, x)
    h1t = _prop(z1t, src, dst, nw, dis, b1)
    Wc = jnp.concatenate([Wmu, Wlv], axis=1)
    bc = jnp.concatenate([bmu, blv])
    z2t = _mm2(Wc, h1t)
    o2t = _prop(z2t, src, dst, nw, dis, bc)
    mu = o2t[:128].T
    logvar = o2t[128:].T
    return (mu, logvar)


# speedup vs baseline: 1.0284x; 1.0284x over previous
import functools

import jax
import jax.numpy as jnp
from jax import lax
from jax.experimental import pallas as pl
from jax.experimental.pallas import tpu as pltpu
from jax.experimental.pallas import tpu_sc as plsc

N = 10000
NPAD = 10240
E = 320000
CH = 3200
NCHUNK = E // CH
FPT = 4
NPASS = 2

_f32 = jnp.float32
_i32 = jnp.int32


def _deg_body(dst_hbm, w_hbm, degp_hbm, ldeg, dstb, wbuf):
    cid = lax.axis_index("c")
    sid = lax.axis_index("s")
    wid = sid * 2 + cid

    def zero_body(i, _):
        ldeg[pl.ds(i * 16, 16)] = jnp.zeros((16,), _f32)
        return 0

    lax.fori_loop(0, NPAD // 16, zero_body, 0)

    per_tile = E // 32
    base = wid * per_tile
    pltpu.sync_copy(dst_hbm.at[pl.ds(base, per_tile)], dstb)
    pltpu.sync_copy(w_hbm.at[pl.ds(base, per_tile)], wbuf)

    def acc_body(s, _):
        dv = dstb[pl.ds(s * 16, 16)]
        wv = wbuf[pl.ds(s * 16, 16)]
        plsc.addupdate_scatter(ldeg, [dv], wv)
        return 0

    lax.fori_loop(0, per_tile // 16, acc_body, 0)
    pltpu.sync_copy(ldeg, degp_hbm.at[wid])


def _norm_body(src_hbm, dst_hbm, w_hbm, dis_hbm, nw_hbm, disb, srcb, dstb, wbuf):
    cid = lax.axis_index("c")
    sid = lax.axis_index("s")
    wid = sid * 2 + cid
    per_tile = E // 32
    base = wid * per_tile

    pltpu.sync_copy(dis_hbm, disb)
    pltpu.sync_copy(src_hbm.at[pl.ds(base, per_tile)], srcb)
    pltpu.sync_copy(dst_hbm.at[pl.ds(base, per_tile)], dstb)
    pltpu.sync_copy(w_hbm.at[pl.ds(base, per_tile)], wbuf)

    def body(s, _):
        sv = srcb[pl.ds(s * 16, 16)]
        dv = dstb[pl.ds(s * 16, 16)]
        wv = wbuf[pl.ds(s * 16, 16)]
        ns = plsc.load_gather(disb, [sv])
        nd = plsc.load_gather(disb, [dv])
        wbuf[pl.ds(s * 16, 16)] = ns * wv * nd
        return 0

    lax.fori_loop(0, per_tile // 16, body, 0)
    pltpu.sync_copy(wbuf, nw_hbm.at[pl.ds(base, per_tile)])


def _prop_body(z_hbm, src_hbm, dst_hbm, w_hbm, dis_hbm, bias_hbm, out_hbm,
               disb, biasb, zb0, zb1, zb2, zb3, ac0, ac1, ac2, ac3,
               srcb0, srcb1, dstb0, dstb1, wb0, wb1, sem0, sem1, semz):
    cid = lax.axis_index("c")
    sid = lax.axis_index("s")
    wid = sid * 2 + cid

    pltpu.sync_copy(dis_hbm, disb)
    pltpu.sync_copy(bias_hbm, biasb)

    zbs = (zb0, zb1, zb2, zb3)
    acs = (ac0, ac1, ac2, ac3)

    srcbs = (srcb0, srcb1)
    dstbs = (dstb0, dstb1)
    wbs = (wb0, wb1)

    def issue(c, slot):
        sem = sem0 if slot == 0 else sem1
        pltpu.async_copy(src_hbm.at[pl.ds(c * CH, CH)], srcbs[slot], sem)
        pltpu.async_copy(dst_hbm.at[pl.ds(c * CH, CH)], dstbs[slot], sem)
        pltpu.async_copy(w_hbm.at[pl.ds(c * CH, CH)], wbs[slot], sem)

    def drain(c, slot):
        sem = sem0 if slot == 0 else sem1
        pltpu.make_async_copy(src_hbm.at[pl.ds(c * CH, CH)], srcbs[slot], sem).wait()
        pltpu.make_async_copy(dst_hbm.at[pl.ds(c * CH, CH)], dstbs[slot], sem).wait()
        pltpu.make_async_copy(w_hbm.at[pl.ds(c * CH, CH)], wbs[slot], sem).wait()

    for p in range(NPASS):
        fb = p * 128 + wid * FPT

        for r in range(FPT):
            pltpu.async_copy(z_hbm.at[fb + r], zbs[r], semz)
        for r in range(FPT):
            pltpu.make_async_copy(z_hbm.at[fb + r], zbs[r], semz).wait()

        bvs = [plsc.load_gather(biasb, [jnp.full((16,), fb + r, _i32)])
               for r in range(FPT)]

        @plsc.parallel_loop(0, N // 16, unroll=4)
        def _init(i):
            dv = disb[pl.ds(i * 16, 16)]
            sn = dv * dv
            for r in range(FPT):
                zv = zbs[r][pl.ds(i * 16, 16)]
                acs[r][pl.ds(i * 16, 16)] = zv * sn + bvs[r]

        issue(0, 0)

        def chunk_pair(g, _):
            for b2 in (0, 1):
                c = g * 2 + b2
                drain(c, b2)

                @pl.when(c + 1 < NCHUNK)
                def _():
                    issue(c + 1, 1 - b2)

                @plsc.parallel_loop(0, CH // 16, unroll=4)
                def _sub(s):
                    sv = srcbs[b2][pl.ds(s * 16, 16)]
                    dv = dstbs[b2][pl.ds(s * 16, 16)]
                    nv = wbs[b2][pl.ds(s * 16, 16)]
                    for r in range(FPT):
                        g_ = plsc.load_gather(zbs[r], [sv])
                        plsc.addupdate_scatter(acs[r], [dv], g_ * nv)
            return 0

        lax.fori_loop(0, NCHUNK // 2, chunk_pair, 0)

        for r in range(FPT):
            pltpu.async_copy(acs[r], out_hbm.at[fb + r], semz)
        for r in range(FPT):
            pltpu.make_async_copy(acs[r], out_hbm.at[fb + r], semz).wait()


_sc_mesh = plsc.VectorSubcoreMesh(core_axis_name="c", subcore_axis_name="s")
_sc_params = pltpu.CompilerParams(needs_layout_passes=False)

_deg = functools.partial(
    pl.kernel,
    mesh=_sc_mesh,
    compiler_params=_sc_params,
    out_type=jax.ShapeDtypeStruct((32, NPAD), _f32),
    scratch_types=[
        pltpu.VMEM((NPAD,), _f32),
        pltpu.VMEM((E // 32,), _i32),
        pltpu.VMEM((E // 32,), _f32),
    ],
)(_deg_body)

_norm = functools.partial(
    pl.kernel,
    mesh=_sc_mesh,
    compiler_params=_sc_params,
    out_type=jax.ShapeDtypeStruct((E,), _f32),
    scratch_types=[
        pltpu.VMEM((NPAD,), _f32),
        pltpu.VMEM((E // 32,), _i32),
        pltpu.VMEM((E // 32,), _i32),
        pltpu.VMEM((E // 32,), _f32),
    ],
)(_norm_body)

_prop = functools.partial(
    pl.kernel,
    mesh=_sc_mesh,
    compiler_params=_sc_params,
    out_type=jax.ShapeDtypeStruct((256, N), _f32),
    scratch_types=[
        pltpu.VMEM((NPAD,), _f32),
        pltpu.VMEM((256,), _f32),
        pltpu.VMEM((N,), _f32), pltpu.VMEM((N,), _f32),
        pltpu.VMEM((N,), _f32), pltpu.VMEM((N,), _f32),
        pltpu.VMEM((N,), _f32), pltpu.VMEM((N,), _f32),
        pltpu.VMEM((N,), _f32), pltpu.VMEM((N,), _f32),
        pltpu.VMEM((CH,), _i32), pltpu.VMEM((CH,), _i32),
        pltpu.VMEM((CH,), _i32), pltpu.VMEM((CH,), _i32),
        pltpu.VMEM((CH,), _f32), pltpu.VMEM((CH,), _f32),
        pltpu.SemaphoreType.DMA,
        pltpu.SemaphoreType.DMA,
        pltpu.SemaphoreType.DMA,
    ],
)(_prop_body)


def _dis_body(degp_ref, dis_ref):
    dis_ref[...] = lax.rsqrt(jnp.sum(degp_ref[...], axis=0) + 1.0)


def _dis(degp):
    return pl.pallas_call(
        _dis_body,
        out_shape=jax.ShapeDtypeStruct((NPAD,), _f32),
    )(degp)


def _mm1_body(w_ref, x_ref, o_ref):
    o_ref[...] = lax.dot_general(
        w_ref[...], x_ref[...], (((0,), (1,)), ((), ())),
        preferred_element_type=_f32)


def _mm2_body(w_ref, h_ref, o_ref):
    h = jnp.maximum(h_ref[...], 0.0)
    o_ref[...] = lax.dot_general(
        w_ref[...], h, (((0,), (0,)), ((), ())),
        preferred_element_type=_f32)


def _mm1(W1, x):
    return pl.pallas_call(
        _mm1_body,
        grid=(2,),
        in_specs=[
            pl.BlockSpec((128, 128), lambda i: (0, i)),
            pl.BlockSpec((N, 128), lambda i: (0, 0)),
        ],
        out_specs=pl.BlockSpec((128, N), lambda i: (i, 0)),
        out_shape=jax.ShapeDtypeStruct((256, N), _f32),
    )(W1, x)


def _mm2(Wc, h1t):
    return pl.pallas_call(
        _mm2_body,
        grid=(2,),
        in_specs=[
            pl.BlockSpec((256, 128), lambda i: (0, i)),
            pl.BlockSpec((256, N), lambda i: (0, 0)),
        ],
        out_specs=pl.BlockSpec((128, N), lambda i: (i, 0)),
        out_shape=jax.ShapeDtypeStruct((256, N), _f32),
    )(Wc, h1t)


def kernel(x, edge_index, edge_attr, W1, b1, Wmu, bmu, Wlv, blv):
    src = edge_index[0]
    dst = edge_index[1]
    w = edge_attr

    degp = _deg(dst, w)
    dis = _dis(degp)
    nw = _norm(src, dst, w, dis)
    z1t = _mm1(W1, x)
    h1t = _prop(z1t, src, dst, nw, dis, b1)
    Wc = jnp.concatenate([Wmu, Wlv], axis=1)
    bc = jnp.concatenate([bmu, blv])
    z2t = _mm2(Wc, h1t)
    o2t = _prop(z2t, src, dst, nw, dis, bc)
    mu = o2t[:128].T
    logvar = o2t[128:].T
    return (mu, logvar)
